# baseline (speedup 1.0000x reference)
import jax
import jax.numpy as jnp
from jax.experimental import pallas as pl
from jax.experimental.pallas import tpu as pltpu

_N = 4096
_D = 64
_K = 4
_BN = 256


def _mixed_op_body(mask_ref, h_ref, w_ref, out_ref):
    acc = jnp.zeros((_N, _D), jnp.float32)
    c = jnp.float32(0.0)
    for k in range(_K):
        wk = w_ref[k]
        acc = acc + jnp.where(wk > 0, wk, 0.0) * h_ref[k]
        c = c + jnp.where(wk > 0, jnp.float32(0.0), wk)
    out_ref[...] = (
        jnp.dot(
            mask_ref[...].astype(jnp.bfloat16),
            acc.astype(jnp.bfloat16),
            preferred_element_type=jnp.float32,
        )
        + c
    )


@jax.jit
def kernel(mask_matrix, h_op_list, weights):
    return pl.pallas_call(
        _mixed_op_body,
        grid=(_N // _BN,),
        in_specs=[
            pl.BlockSpec((_BN, _N), lambda i: (i, 0)),
            pl.BlockSpec((_K, _N, _D), lambda i: (0, 0, 0)),
            pl.BlockSpec(memory_space=pltpu.SMEM),
        ],
        out_specs=pl.BlockSpec((_BN, _D), lambda i: (i, 0)),
        out_shape=jax.ShapeDtypeStruct((_N, _D), jnp.float32),
        compiler_params=pltpu.CompilerParams(
            dimension_semantics=("parallel",),
        ),
    )(mask_matrix, h_op_list, weights)


# speedup vs baseline: 1.1014x; 1.1014x over previous
import jax
import jax.numpy as jnp
from jax.experimental import pallas as pl
from jax.experimental.pallas import tpu as pltpu

_N = 4096
_D = 64
_K = 4
_BN = 1024


def _mixed_op_body(mask_ref, h_ref, w_ref, out_ref):
    acc = jnp.zeros((_N, _D), jnp.float32)
    c = jnp.float32(0.0)
    for k in range(_K):
        wk = w_ref[k]
        acc = acc + jnp.where(wk > 0, wk, 0.0) * h_ref[k]
        c = c + jnp.where(wk > 0, jnp.float32(0.0), wk)
    out_ref[...] = (
        jnp.dot(
            mask_ref[...].astype(jnp.bfloat16),
            acc.astype(jnp.bfloat16),
            preferred_element_type=jnp.float32,
        )
        + c
    )


@jax.jit
def kernel(mask_matrix, h_op_list, weights):
    return pl.pallas_call(
        _mixed_op_body,
        grid=(_N // _BN,),
        in_specs=[
            pl.BlockSpec((_BN, _N), lambda i: (i, 0)),
            pl.BlockSpec((_K, _N, _D), lambda i: (0, 0, 0)),
            pl.BlockSpec(memory_space=pltpu.SMEM),
        ],
        out_specs=pl.BlockSpec((_BN, _D), lambda i: (i, 0)),
        out_shape=jax.ShapeDtypeStruct((_N, _D), jnp.float32),
        compiler_params=pltpu.CompilerParams(
            dimension_semantics=("parallel",),
        ),
    )(mask_matrix, h_op_list, weights)


# speedup vs baseline: 1.1205x; 1.0173x over previous
import jax
import jax.numpy as jnp
from jax.experimental import pallas as pl
from jax.experimental.pallas import tpu as pltpu

_N = 4096
_D = 64
_K = 4
_CH = 256
_NSTEP = _N // _CH
_NBUF = 10
_LOOK = _NBUF - 2


def _chunk_copy(mask_hbm, mbuf, sems, chunk, slot):
    return pltpu.make_async_copy(
        mask_hbm.at[pl.ds(chunk * _CH, _CH), :],
        mbuf.at[slot],
        sems.at[slot],
    )


def _mixed_op_body(mask_hbm, h_ref, w_ref, out_ref, mbuf, hc_ref, sems):
    i = pl.program_id(0)

    @pl.when(i == 0)
    def _prologue():
        for j in range(_LOOK):
            _chunk_copy(mask_hbm, mbuf, sems, j, j).start()
        acc = jnp.zeros((_N, _D), jnp.float32)
        for k in range(_K):
            wk = w_ref[k]
            acc = acc + jnp.where(wk > 0, wk, 0.0) * h_ref[k]
        hc_ref[...] = acc.astype(jnp.bfloat16)

    _chunk_copy(mask_hbm, mbuf, sems, i, i % _NBUF).wait()

    nxt = i + _LOOK

    @pl.when(nxt < _NSTEP)
    def _prefetch():
        _chunk_copy(mask_hbm, mbuf, sems, nxt, nxt % _NBUF).start()

    c = jnp.float32(0.0)
    for k in range(_K):
        wk = w_ref[k]
        c = c + jnp.where(wk > 0, jnp.float32(0.0), wk)
    out_ref[...] = (
        jnp.dot(
            mbuf[i % _NBUF].astype(jnp.bfloat16),
            hc_ref[...],
            preferred_element_type=jnp.float32,
        )
        + c
    )


@jax.jit
def kernel(mask_matrix, h_op_list, weights):
    return pl.pallas_call(
        _mixed_op_body,
        grid=(_NSTEP,),
        in_specs=[
            pl.BlockSpec(memory_space=pltpu.HBM),
            pl.BlockSpec((_K, _N, _D), lambda i: (0, 0, 0)),
            pl.BlockSpec(memory_space=pltpu.SMEM),
        ],
        out_specs=pl.BlockSpec((_CH, _D), lambda i: (i, 0)),
        out_shape=jax.ShapeDtypeStruct((_N, _D), jnp.float32),
        scratch_shapes=[
            pltpu.VMEM((_NBUF, _CH, _N), jnp.float32),
            pltpu.VMEM((_N, _D), jnp.bfloat16),
            pltpu.SemaphoreType.DMA((_NBUF,)),
        ],
    )(mask_matrix, h_op_list, weights)
